# baseline (speedup 1.0000x reference)
import functools

import jax
import jax.numpy as jnp
from jax import lax
from jax.experimental import pallas as pl
from jax.experimental.pallas import tpu as pltpu
from jax.experimental.pallas import tpu_sc as plsc

L = 512
DS = 384
DP = 64
VP = 32
TI = 64
TJ2 = 32
RELPAD = 1064


NTOK = 1024
TPW = 32
NCH = DS // 16


def _seq_sc_kernel(x_hbm, tab_hbm, pos_hbm, g_hbm, b_hbm, out_hbm,
                   idx_v, rows_v, pos_v, g_v, b_v, sem):
    wid = lax.axis_index("s") * 2 + lax.axis_index("c")
    base = wid * TPW
    pltpu.sync_copy(x_hbm.at[pl.ds(base, TPW)], idx_v)
    cp = pltpu.async_copy(tab_hbm.at[idx_v], rows_v, sem)
    pltpu.sync_copy(pos_hbm.at[pl.ds(base % L, TPW)], pos_v)
    pltpu.sync_copy(g_hbm, g_v)
    pltpu.sync_copy(b_hbm, b_v)
    cp.wait()

    lanes = lax.iota(jnp.int32, 16)

    def _allsum(v):
        for kk in (8, 4, 2, 1):
            v = v + jnp.take(v, jnp.bitwise_xor(lanes, kk))
        return v

    def token(i, _):
        zero = jnp.zeros((16,), jnp.float32)

        def acc(k, carry):
            s, s2 = carry
            v = rows_v[i, pl.ds(k * 16, 16)] + pos_v[i, pl.ds(k * 16, 16)]
            rows_v[i, pl.ds(k * 16, 16)] = v
            return (s + v, s2 + v * v)

        s, s2 = lax.fori_loop(0, NCH, acc, (zero, zero))
        m = _allsum(s) * (1.0 / DS)
        var = _allsum(s2) * (1.0 / DS) - m * m
        xv = var + 1e-5
        yi = jnp.full((16,), 0x5F3759DF, jnp.int32) - (
            lax.bitcast_convert_type(xv, jnp.int32) >> 1)
        y = lax.bitcast_convert_type(yi, jnp.float32)
        for _ in range(4):
            y = y * (1.5 - 0.5 * xv * y * y)

        def norm(k, _):
            v = rows_v[i, pl.ds(k * 16, 16)]
            rows_v[i, pl.ds(k * 16, 16)] = (
                (v - m) * y * g_v[pl.ds(k * 16, 16)] + b_v[pl.ds(k * 16, 16)])
            return 0

        return lax.fori_loop(0, NCH, norm, 0)

    lax.fori_loop(0, TPW, token, 0)
    pltpu.sync_copy(rows_v, out_hbm.at[pl.ds(base, TPW)])


def _seq_call(x_flat, seq_aa_emb, abs_pos_emb, g, b):
    mesh = plsc.VectorSubcoreMesh(core_axis_name="c", subcore_axis_name="s")
    fn = functools.partial(
        pl.kernel,
        mesh=mesh,
        out_type=jax.ShapeDtypeStruct((NTOK, DS), jnp.float32),
        scratch_types=[
            pltpu.VMEM((TPW,), jnp.int32),
            pltpu.VMEM((TPW, DS), jnp.float32),
            pltpu.VMEM((TPW, DS), jnp.float32),
            pltpu.VMEM((DS,), jnp.float32),
            pltpu.VMEM((DS,), jnp.float32),
            pltpu.SemaphoreType.DMA,
        ],
    )(_seq_sc_kernel)
    return fn(x_flat, seq_aa_emb, abs_pos_emb, g, b)


def _pair_kernel(xi_ref, xj_ref, ti_ref, tj_ref, rel_a_ref,
                 g_ref, b_ref, out_ref, m2_scr, sbd_scr, rel_scr, eye_scr):
    bi = pl.program_id(0)
    bj = pl.program_id(1)
    b = pl.program_id(2)
    NR = TI * TJ2

    @pl.when((bi == 0) & (bj == 0) & (b == 0))
    def _build_static():
        r = jax.lax.broadcasted_iota(jnp.int32, (NR, 128), 0)
        s = jax.lax.broadcasted_iota(jnp.int32, (NR, 128), 1)
        m2_scr[...] = (s == (r // TJ2) - (r % TJ2) + 32
                       ).astype(jnp.float32).astype(jnp.bfloat16)
        a = jax.lax.broadcasted_iota(jnp.int32, (128, 128), 0)
        c = jax.lax.broadcasted_iota(jnp.int32, (128, 128), 1)
        sbd_scr[...] = ((a // 64 == c // 64).astype(jnp.float32)
                        * (1.0 / 64.0)).astype(jnp.bfloat16)
        e0 = jax.lax.broadcasted_iota(jnp.int32, (2 * TJ2, 2 * TJ2), 0)
        e1 = jax.lax.broadcasted_iota(jnp.int32, (2 * TJ2, 2 * TJ2), 1)
        eye_scr[...] = (e0 == e1).astype(jnp.float32).astype(jnp.bfloat16)

    @pl.when(b == 0)
    def _build_rel():
        for t in (0, 1):
            start = 64 * (bi - 2 * bj - t) + 480
            w_l = rel_a_ref[pl.ds(start, 128), :]
            w_r = rel_a_ref[pl.ds(start - 32, 128), :]
            waug = jnp.concatenate([w_l, w_r], axis=1)
            rel_scr[t] = jnp.dot(
                m2_scr[...], waug,
                preferred_element_type=jnp.float32).astype(jnp.bfloat16)

    xi = xi_ref[0]
    oh_i = (xi == jax.lax.broadcasted_iota(jnp.int32, (TI, VP), 1)
            ).astype(jnp.float32).astype(jnp.bfloat16)
    pi = jnp.dot(oh_i, ti_ref[...],
                 preferred_element_type=jnp.float32).astype(jnp.bfloat16)

    subs = []
    for t in (0, 1):
        xj = xj_ref[0, pl.ds(64 * t, 64)]
        xpb = jnp.concatenate(
            [jnp.broadcast_to(xj[0:TJ2], (TJ2, VP)),
             jnp.broadcast_to(xj[TJ2:2 * TJ2], (TJ2, VP))], axis=1)
        vmod = jax.lax.broadcasted_iota(jnp.int32, (TJ2, 2 * VP), 1) % VP
        oh_j = (xpb == vmod).astype(jnp.float32).astype(jnp.bfloat16)
        pj = jnp.dot(oh_j, tj_ref[...],
                     preferred_element_type=jnp.float32).astype(jnp.bfloat16)

        e3 = (rel_scr[t].reshape(TI, TJ2, 128)
              + pi[:, None, :] + pj[None, :, :])
        ebf = e3.reshape(NR, 128)
        m = jnp.dot(ebf, sbd_scr[...], preferred_element_type=jnp.float32)
        q = jnp.dot(ebf * ebf, sbd_scr[...],
                    preferred_element_type=jnp.float32)
        var = q - m * m
        scale = jax.lax.rsqrt(var + 1e-5) * g_ref[0]
        scale_bf = scale.astype(jnp.bfloat16)
        shift_bf = (b_ref[0] - m * scale).astype(jnp.bfloat16)
        out = ebf * scale_bf + shift_bf
        e_un = jnp.concatenate(
            [out[:, 0:DP].reshape(TI, TJ2, DP),
             out[:, DP:128].reshape(TI, TJ2, DP)], axis=1)
        out_ref[0, :, :, pl.ds(64 * t, 64)] = jax.lax.dot_general(
            e_un, eye_scr[...],
            dimension_numbers=(((1,), (0,)), ((), ())),
            preferred_element_type=jnp.float32)


def kernel(x, seq_aa_emb, abs_pos_emb, pair_aa_emb_i, pair_aa_emb_j,
           relpos_emb, seq_ln_g, seq_ln_b, pair_ln_g, pair_ln_b):
    B, Lx = x.shape
    xi = x.astype(jnp.int32)
    x3 = xi.reshape(B, Lx, 1)

    def pad_tab(t):
        return jnp.concatenate(
            [t, jnp.zeros((VP - t.shape[0], t.shape[1]), t.dtype)], axis=0)

    ti_tab = jnp.tile(pad_tab(pair_aa_emb_i), (1, 2)).astype(jnp.bfloat16)
    tjp = pad_tab(pair_aa_emb_j)
    z = jnp.zeros((VP, DP), tjp.dtype)
    tj_tab = jnp.concatenate(
        [jnp.concatenate([tjp, z], axis=1),
         jnp.concatenate([z, tjp], axis=1)], axis=0).astype(jnp.bfloat16)
    rel_a = jnp.concatenate(
        [jnp.zeros((1, DP), relpos_emb.dtype), relpos_emb,
         jnp.zeros((RELPAD - 1 - relpos_emb.shape[0], DP),
                   relpos_emb.dtype)], axis=0).astype(jnp.bfloat16)

    seq_repr = _seq_call(
        xi.reshape(B * Lx), seq_aa_emb, abs_pos_emb, seq_ln_g, seq_ln_b
    ).reshape(B, Lx, DS)

    g2 = jnp.tile(pair_ln_g.reshape(1, DP), (1, 2))
    b2 = jnp.tile(pair_ln_b.reshape(1, DP), (1, 2))

    Gi, Gj = Lx // TI, Lx // 128
    pair2 = pl.pallas_call(
        _pair_kernel,
        grid=(Gi, Gj, B),
        in_specs=[
            pl.BlockSpec((1, TI, 1), lambda bi, bj, b: (b, bi, 0)),
            pl.BlockSpec((1, 128, 1), lambda bi, bj, b: (b, bj, 0)),
            pl.BlockSpec((VP, 128), lambda bi, bj, b: (0, 0)),
            pl.BlockSpec((2 * VP, 128), lambda bi, bj, b: (0, 0)),
            pl.BlockSpec((RELPAD, DP), lambda bi, bj, b: (0, 0)),
            pl.BlockSpec((1, 128), lambda bi, bj, b: (0, 0)),
            pl.BlockSpec((1, 128), lambda bi, bj, b: (0, 0)),
        ],
        out_specs=pl.BlockSpec((1, TI, DP, 128),
                               lambda bi, bj, b: (b, bi, 0, bj)),
        out_shape=jax.ShapeDtypeStruct((B, Lx, DP, Lx), jnp.float32),
        scratch_shapes=[
            pltpu.VMEM((TI * TJ2, 128), jnp.bfloat16),
            pltpu.VMEM((128, 128), jnp.bfloat16),
            pltpu.VMEM((2, TI * TJ2, 128), jnp.bfloat16),
            pltpu.VMEM((2 * TJ2, 2 * TJ2), jnp.bfloat16),
        ],
    )(x3, x3, ti_tab, tj_tab, rel_a, g2, b2)

    return (seq_repr, jnp.transpose(pair2, (0, 1, 3, 2)))


# speedup vs baseline: 1.2678x; 1.2678x over previous
import functools

import jax
import jax.numpy as jnp
from jax import lax
from jax.experimental import pallas as pl
from jax.experimental.pallas import tpu as pltpu
from jax.experimental.pallas import tpu_sc as plsc

L = 512
DS = 384
DP = 64
VP = 32
TI = 64
TJ2 = 32
RELPAD = 1064


NTOK = 1024
TPW = 32
NCH = DS // 16


def _seq_sc_kernel(x_hbm, tab_hbm, pos_hbm, g_hbm, b_hbm, out_hbm,
                   idx_v, rows_v, pos_v, g_v, b_v, sem):
    wid = lax.axis_index("s") * 2 + lax.axis_index("c")
    base = wid * TPW
    pltpu.sync_copy(x_hbm.at[pl.ds(base, TPW)], idx_v)
    cp = pltpu.async_copy(tab_hbm.at[idx_v], rows_v, sem)
    pltpu.sync_copy(pos_hbm.at[pl.ds(base % L, TPW)], pos_v)
    pltpu.sync_copy(g_hbm, g_v)
    pltpu.sync_copy(b_hbm, b_v)
    cp.wait()

    lanes = lax.iota(jnp.int32, 16)

    def _allsum(v):
        for kk in (8, 4, 2, 1):
            v = v + jnp.take(v, jnp.bitwise_xor(lanes, kk))
        return v

    def token(i, _):
        zero = jnp.zeros((16,), jnp.float32)

        def acc(k, carry):
            s, s2 = carry
            v = rows_v[i, pl.ds(k * 16, 16)] + pos_v[i, pl.ds(k * 16, 16)]
            rows_v[i, pl.ds(k * 16, 16)] = v
            return (s + v, s2 + v * v)

        s, s2 = lax.fori_loop(0, NCH, acc, (zero, zero))
        m = _allsum(s) * (1.0 / DS)
        var = _allsum(s2) * (1.0 / DS) - m * m
        xv = var + 1e-5
        yi = jnp.full((16,), 0x5F3759DF, jnp.int32) - (
            lax.bitcast_convert_type(xv, jnp.int32) >> 1)
        y = lax.bitcast_convert_type(yi, jnp.float32)
        for _ in range(4):
            y = y * (1.5 - 0.5 * xv * y * y)

        def norm(k, _):
            v = rows_v[i, pl.ds(k * 16, 16)]
            rows_v[i, pl.ds(k * 16, 16)] = (
                (v - m) * y * g_v[pl.ds(k * 16, 16)] + b_v[pl.ds(k * 16, 16)])
            return 0

        return lax.fori_loop(0, NCH, norm, 0)

    lax.fori_loop(0, TPW, token, 0)
    pltpu.sync_copy(rows_v, out_hbm.at[pl.ds(base, TPW)])


def _seq_call(x_flat, seq_aa_emb, abs_pos_emb, g, b):
    mesh = plsc.VectorSubcoreMesh(core_axis_name="c", subcore_axis_name="s")
    fn = functools.partial(
        pl.kernel,
        mesh=mesh,
        out_type=jax.ShapeDtypeStruct((NTOK, DS), jnp.float32),
        scratch_types=[
            pltpu.VMEM((TPW,), jnp.int32),
            pltpu.VMEM((TPW, DS), jnp.float32),
            pltpu.VMEM((TPW, DS), jnp.float32),
            pltpu.VMEM((DS,), jnp.float32),
            pltpu.VMEM((DS,), jnp.float32),
            pltpu.SemaphoreType.DMA,
        ],
    )(_seq_sc_kernel)
    return fn(x_flat, seq_aa_emb, abs_pos_emb, g, b)


def _pair_kernel(xi_ref, xj_ref, ti_ref, tj_ref, rel_a_ref,
                 g_ref, b_ref, out_ref, m2_scr, sbd_scr, rel_scr, eye_scr):
    bi = pl.program_id(0)
    bj = pl.program_id(1)
    b = pl.program_id(2)
    NR = TI * TJ2

    @pl.when((bi == 0) & (bj == 0) & (b == 0))
    def _build_static():
        r = jax.lax.broadcasted_iota(jnp.int32, (NR, 128), 0)
        s = jax.lax.broadcasted_iota(jnp.int32, (NR, 128), 1)
        m2_scr[...] = (s == (r // TJ2) - (r % TJ2) + 32
                       ).astype(jnp.float32).astype(jnp.bfloat16)
        a = jax.lax.broadcasted_iota(jnp.int32, (128, 128), 0)
        c = jax.lax.broadcasted_iota(jnp.int32, (128, 128), 1)
        sbd_scr[...] = ((a // 64 == c // 64).astype(jnp.float32)
                        * (1.0 / 64.0)).astype(jnp.bfloat16)
        e0 = jax.lax.broadcasted_iota(jnp.int32, (128, 128), 0)
        e1 = jax.lax.broadcasted_iota(jnp.int32, (128, 128), 1)
        eye_scr[...] = (e0 == e1).astype(jnp.float32).astype(jnp.bfloat16)

    @pl.when(b == 0)
    def _build_rel():
        for t in (0, 1):
            start = 64 * (bi - 2 * bj - t) + 480
            w_l = rel_a_ref[pl.ds(start, 128), :]
            w_r = rel_a_ref[pl.ds(start - 32, 128), :]
            waug = jnp.concatenate([w_l, w_r], axis=1)
            rel_scr[t] = jnp.dot(
                m2_scr[...], waug,
                preferred_element_type=jnp.float32).astype(jnp.bfloat16)

    xi = xi_ref[0]
    oh_i = (xi == jax.lax.broadcasted_iota(jnp.int32, (TI, VP), 1)
            ).astype(jnp.float32).astype(jnp.bfloat16)
    pi = jnp.dot(oh_i, ti_ref[...],
                 preferred_element_type=jnp.float32).astype(jnp.bfloat16)

    e_uns = []
    for t in (0, 1):
        xj = xj_ref[0, pl.ds(64 * t, 64)]
        xpb = jnp.concatenate(
            [jnp.broadcast_to(xj[0:TJ2], (TJ2, VP)),
             jnp.broadcast_to(xj[TJ2:2 * TJ2], (TJ2, VP))], axis=1)
        vmod = jax.lax.broadcasted_iota(jnp.int32, (TJ2, 2 * VP), 1) % VP
        oh_j = (xpb == vmod).astype(jnp.float32).astype(jnp.bfloat16)
        pj = jnp.dot(oh_j, tj_ref[...],
                     preferred_element_type=jnp.float32).astype(jnp.bfloat16)

        e3 = (rel_scr[t].reshape(TI, TJ2, 128)
              + pi[:, None, :] + pj[None, :, :])
        ebf = e3.reshape(NR, 128)
        m = jnp.dot(ebf, sbd_scr[...], preferred_element_type=jnp.float32)
        q = jnp.dot(ebf * ebf, sbd_scr[...],
                    preferred_element_type=jnp.float32)
        var = q - m * m
        scale = jax.lax.rsqrt(var + 1e-5) * g_ref[0]
        scale_bf = scale.astype(jnp.bfloat16)
        shift_bf = (b_ref[0] - m * scale).astype(jnp.bfloat16)
        out = ebf * scale_bf + shift_bf
        e_uns.append(jnp.concatenate(
            [out[:, 0:DP].reshape(TI, TJ2, DP),
             out[:, DP:128].reshape(TI, TJ2, DP)], axis=1))

    e_un2 = jnp.concatenate(e_uns, axis=1)
    out_ref[0] = jax.lax.dot_general(
        e_un2, eye_scr[...],
        dimension_numbers=(((1,), (0,)), ((), ())),
        preferred_element_type=jnp.float32)


def kernel(x, seq_aa_emb, abs_pos_emb, pair_aa_emb_i, pair_aa_emb_j,
           relpos_emb, seq_ln_g, seq_ln_b, pair_ln_g, pair_ln_b):
    B, Lx = x.shape
    xi = x.astype(jnp.int32)
    x3 = xi.reshape(B, Lx, 1)

    def pad_tab(t):
        return jnp.concatenate(
            [t, jnp.zeros((VP - t.shape[0], t.shape[1]), t.dtype)], axis=0)

    ti_tab = jnp.tile(pad_tab(pair_aa_emb_i), (1, 2)).astype(jnp.bfloat16)
    tjp = pad_tab(pair_aa_emb_j)
    z = jnp.zeros((VP, DP), tjp.dtype)
    tj_tab = jnp.concatenate(
        [jnp.concatenate([tjp, z], axis=1),
         jnp.concatenate([z, tjp], axis=1)], axis=0).astype(jnp.bfloat16)
    rel_a = jnp.concatenate(
        [jnp.zeros((1, DP), relpos_emb.dtype), relpos_emb,
         jnp.zeros((RELPAD - 1 - relpos_emb.shape[0], DP),
                   relpos_emb.dtype)], axis=0).astype(jnp.bfloat16)

    seq_repr = _seq_call(
        xi.reshape(B * Lx), seq_aa_emb, abs_pos_emb, seq_ln_g, seq_ln_b
    ).reshape(B, Lx, DS)

    g2 = jnp.tile(pair_ln_g.reshape(1, DP), (1, 2))
    b2 = jnp.tile(pair_ln_b.reshape(1, DP), (1, 2))

    Gi, Gj = Lx // TI, Lx // 128
    pair2 = pl.pallas_call(
        _pair_kernel,
        grid=(Gi, Gj, B),
        in_specs=[
            pl.BlockSpec((1, TI, 1), lambda bi, bj, b: (b, bi, 0)),
            pl.BlockSpec((1, 128, 1), lambda bi, bj, b: (b, bj, 0)),
            pl.BlockSpec((VP, 128), lambda bi, bj, b: (0, 0)),
            pl.BlockSpec((2 * VP, 128), lambda bi, bj, b: (0, 0)),
            pl.BlockSpec((RELPAD, DP), lambda bi, bj, b: (0, 0)),
            pl.BlockSpec((1, 128), lambda bi, bj, b: (0, 0)),
            pl.BlockSpec((1, 128), lambda bi, bj, b: (0, 0)),
        ],
        out_specs=pl.BlockSpec((1, TI, DP, 128),
                               lambda bi, bj, b: (b, bi, 0, bj)),
        out_shape=jax.ShapeDtypeStruct((B, Lx, DP, Lx), jnp.float32),
        scratch_shapes=[
            pltpu.VMEM((TI * TJ2, 128), jnp.bfloat16),
            pltpu.VMEM((128, 128), jnp.bfloat16),
            pltpu.VMEM((2, TI * TJ2, 128), jnp.bfloat16),
            pltpu.VMEM((128, 128), jnp.bfloat16),
        ],
    )(x3, x3, ti_tab, tj_tab, rel_a, g2, b2)

    return (seq_repr, jnp.transpose(pair2, (0, 1, 3, 2)))


# speedup vs baseline: 1.3212x; 1.0421x over previous
import functools

import jax
import jax.numpy as jnp
from jax import lax
from jax.experimental import pallas as pl
from jax.experimental.pallas import tpu as pltpu
from jax.experimental.pallas import tpu_sc as plsc

L = 512
DS = 384
DP = 64
VP = 32
TI = 64
TJ2 = 32
RELPAD = 1064


NTOK = 1024
TPW = 32
NCH = DS // 16


def _seq_sc_kernel(x_hbm, tab_hbm, pos_hbm, g_hbm, b_hbm, out_hbm,
                   idx_v, rows_v, pos_v, g_v, b_v, sem):
    wid = lax.axis_index("s") * 2 + lax.axis_index("c")
    base = wid * TPW
    pltpu.sync_copy(x_hbm.at[pl.ds(base, TPW)], idx_v)
    cp = pltpu.async_copy(tab_hbm.at[idx_v], rows_v, sem)
    pltpu.sync_copy(pos_hbm.at[pl.ds(base % L, TPW)], pos_v)
    pltpu.sync_copy(g_hbm, g_v)
    pltpu.sync_copy(b_hbm, b_v)
    cp.wait()

    lanes = lax.iota(jnp.int32, 16)

    def _allsum(v):
        for kk in (8, 4, 2, 1):
            v = v + jnp.take(v, jnp.bitwise_xor(lanes, kk))
        return v

    def token(i, _):
        zero = jnp.zeros((16,), jnp.float32)

        def acc(k, carry):
            s, s2 = carry
            v = rows_v[i, pl.ds(k * 16, 16)] + pos_v[i, pl.ds(k * 16, 16)]
            rows_v[i, pl.ds(k * 16, 16)] = v
            return (s + v, s2 + v * v)

        s, s2 = lax.fori_loop(0, NCH, acc, (zero, zero))
        m = _allsum(s) * (1.0 / DS)
        var = _allsum(s2) * (1.0 / DS) - m * m
        xv = var + 1e-5
        yi = jnp.full((16,), 0x5F3759DF, jnp.int32) - (
            lax.bitcast_convert_type(xv, jnp.int32) >> 1)
        y = lax.bitcast_convert_type(yi, jnp.float32)
        for _ in range(4):
            y = y * (1.5 - 0.5 * xv * y * y)

        def norm(k, _):
            v = rows_v[i, pl.ds(k * 16, 16)]
            rows_v[i, pl.ds(k * 16, 16)] = (
                (v - m) * y * g_v[pl.ds(k * 16, 16)] + b_v[pl.ds(k * 16, 16)])
            return 0

        return lax.fori_loop(0, NCH, norm, 0)

    lax.fori_loop(0, TPW, token, 0)
    pltpu.sync_copy(rows_v, out_hbm.at[pl.ds(base, TPW)])


def _seq_call(x_flat, seq_aa_emb, abs_pos_emb, g, b):
    mesh = plsc.VectorSubcoreMesh(core_axis_name="c", subcore_axis_name="s")
    fn = functools.partial(
        pl.kernel,
        mesh=mesh,
        out_type=jax.ShapeDtypeStruct((NTOK, DS), jnp.float32),
        scratch_types=[
            pltpu.VMEM((TPW,), jnp.int32),
            pltpu.VMEM((TPW, DS), jnp.float32),
            pltpu.VMEM((TPW, DS), jnp.float32),
            pltpu.VMEM((DS,), jnp.float32),
            pltpu.VMEM((DS,), jnp.float32),
            pltpu.SemaphoreType.DMA,
        ],
    )(_seq_sc_kernel)
    return fn(x_flat, seq_aa_emb, abs_pos_emb, g, b)


def _pair_kernel(xi_ref, xj_ref, ti_ref, tj_ref, rel_a_ref,
                 g_ref, b_ref, out_ref, m2_scr, sbd_scr, rel_scr, eye_scr):
    bi = pl.program_id(0)
    bj = pl.program_id(1)
    b = pl.program_id(2)
    NR = TI * TJ2

    @pl.when((bi == 0) & (bj == 0) & (b == 0))
    def _build_static():
        r = jax.lax.broadcasted_iota(jnp.int32, (NR, 128), 0)
        s = jax.lax.broadcasted_iota(jnp.int32, (NR, 128), 1)
        m2_scr[...] = (s == (r // TJ2) - (r % TJ2) + 32
                       ).astype(jnp.float32).astype(jnp.bfloat16)
        a = jax.lax.broadcasted_iota(jnp.int32, (128, 128), 0)
        c = jax.lax.broadcasted_iota(jnp.int32, (128, 128), 1)
        sbd_scr[...] = ((a // 64 == c // 64).astype(jnp.float32)
                        * (1.0 / 64.0)).astype(jnp.bfloat16)
        e0 = jax.lax.broadcasted_iota(jnp.int32, (256, 256), 0)
        e1 = jax.lax.broadcasted_iota(jnp.int32, (256, 256), 1)
        eye_scr[...] = (e0 == e1).astype(jnp.float32).astype(jnp.bfloat16)

    @pl.when(b == 0)
    def _build_rel():
        for t in (0, 1, 2, 3):
            start = 64 * (bi - 4 * bj - t) + 480
            w_l = rel_a_ref[pl.ds(start, 128), :]
            w_r = rel_a_ref[pl.ds(start - 32, 128), :]
            waug = jnp.concatenate([w_l, w_r], axis=1)
            rel_scr[t] = jnp.dot(
                m2_scr[...], waug,
                preferred_element_type=jnp.float32).astype(jnp.bfloat16)

    xi = xi_ref[0]
    oh_i = (xi == jax.lax.broadcasted_iota(jnp.int32, (TI, VP), 1)
            ).astype(jnp.float32).astype(jnp.bfloat16)
    pi = jnp.dot(oh_i, ti_ref[...],
                 preferred_element_type=jnp.float32).astype(jnp.bfloat16)

    e_uns = []
    for t in (0, 1, 2, 3):
        xj = xj_ref[0, pl.ds(64 * t, 64)]
        xpb = jnp.concatenate(
            [jnp.broadcast_to(xj[0:TJ2], (TJ2, VP)),
             jnp.broadcast_to(xj[TJ2:2 * TJ2], (TJ2, VP))], axis=1)
        vmod = jax.lax.broadcasted_iota(jnp.int32, (TJ2, 2 * VP), 1) % VP
        oh_j = (xpb == vmod).astype(jnp.float32).astype(jnp.bfloat16)
        pj = jnp.dot(oh_j, tj_ref[...],
                     preferred_element_type=jnp.float32).astype(jnp.bfloat16)

        e3 = (rel_scr[t].reshape(TI, TJ2, 128)
              + pi[:, None, :] + pj[None, :, :])
        ebf = e3.reshape(NR, 128)
        m = jnp.dot(ebf, sbd_scr[...], preferred_element_type=jnp.float32)
        q = jnp.dot(ebf * ebf, sbd_scr[...],
                    preferred_element_type=jnp.float32)
        var = q - m * m
        scale = jax.lax.rsqrt(var + 1e-5) * g_ref[0]
        scale_bf = scale.astype(jnp.bfloat16)
        shift_bf = (b_ref[0] - m * scale).astype(jnp.bfloat16)
        out = ebf * scale_bf + shift_bf
        e_uns.append(jnp.concatenate(
            [out[:, 0:DP].reshape(TI, TJ2, DP),
             out[:, DP:128].reshape(TI, TJ2, DP)], axis=1))

    e_un2 = jnp.concatenate(e_uns, axis=1)
    out_ref[0] = jax.lax.dot_general(
        e_un2, eye_scr[...],
        dimension_numbers=(((1,), (0,)), ((), ())),
        preferred_element_type=jnp.float32)


def kernel(x, seq_aa_emb, abs_pos_emb, pair_aa_emb_i, pair_aa_emb_j,
           relpos_emb, seq_ln_g, seq_ln_b, pair_ln_g, pair_ln_b):
    B, Lx = x.shape
    xi = x.astype(jnp.int32)
    x3 = xi.reshape(B, Lx, 1)

    def pad_tab(t):
        return jnp.concatenate(
            [t, jnp.zeros((VP - t.shape[0], t.shape[1]), t.dtype)], axis=0)

    ti_tab = jnp.tile(pad_tab(pair_aa_emb_i), (1, 2)).astype(jnp.bfloat16)
    tjp = pad_tab(pair_aa_emb_j)
    z = jnp.zeros((VP, DP), tjp.dtype)
    tj_tab = jnp.concatenate(
        [jnp.concatenate([tjp, z], axis=1),
         jnp.concatenate([z, tjp], axis=1)], axis=0).astype(jnp.bfloat16)
    rel_a = jnp.concatenate(
        [jnp.zeros((1, DP), relpos_emb.dtype), relpos_emb,
         jnp.zeros((RELPAD - 1 - relpos_emb.shape[0], DP),
                   relpos_emb.dtype)], axis=0).astype(jnp.bfloat16)

    seq_repr = _seq_call(
        xi.reshape(B * Lx), seq_aa_emb, abs_pos_emb, seq_ln_g, seq_ln_b
    ).reshape(B, Lx, DS)

    g2 = jnp.tile(pair_ln_g.reshape(1, DP), (1, 2))
    b2 = jnp.tile(pair_ln_b.reshape(1, DP), (1, 2))

    Gi, Gj = Lx // TI, Lx // 256
    pair2 = pl.pallas_call(
        _pair_kernel,
        grid=(Gi, Gj, B),
        in_specs=[
            pl.BlockSpec((1, TI, 1), lambda bi, bj, b: (b, bi, 0)),
            pl.BlockSpec((1, 256, 1), lambda bi, bj, b: (b, bj, 0)),
            pl.BlockSpec((VP, 128), lambda bi, bj, b: (0, 0)),
            pl.BlockSpec((2 * VP, 128), lambda bi, bj, b: (0, 0)),
            pl.BlockSpec((RELPAD, DP), lambda bi, bj, b: (0, 0)),
            pl.BlockSpec((1, 128), lambda bi, bj, b: (0, 0)),
            pl.BlockSpec((1, 128), lambda bi, bj, b: (0, 0)),
        ],
        out_specs=pl.BlockSpec((1, TI, DP, 256),
                               lambda bi, bj, b: (b, bi, 0, bj)),
        out_shape=jax.ShapeDtypeStruct((B, Lx, DP, Lx), jnp.float32),
        scratch_shapes=[
            pltpu.VMEM((TI * TJ2, 128), jnp.bfloat16),
            pltpu.VMEM((128, 128), jnp.bfloat16),
            pltpu.VMEM((4, TI * TJ2, 128), jnp.bfloat16),
            pltpu.VMEM((256, 256), jnp.bfloat16),
        ],
    )(x3, x3, ti_tab, tj_tab, rel_a, g2, b2)

    return (seq_repr, jnp.transpose(pair2, (0, 1, 3, 2)))
